# baseline (speedup 1.0000x reference)
import functools

import jax
import jax.numpy as jnp
from jax import lax
from jax.experimental import pallas as pl
from jax.experimental.pallas import tpu as pltpu
from jax.experimental.pallas import tpu_sc as plsc

ROWS = 128
SCAN = 100000
LANES = 16
NUM_CORES = 2
NUM_SUBCORES = 16
NUM_WORKERS = NUM_CORES * NUM_SUBCORES

CGROUPS = ROWS // LANES
TILE_R = 8
CHUNK_TR = 46
CHUNK_ROWS = CHUNK_TR * TILE_R
NCHUNKS = 4
TR_PER_W = CHUNK_TR * NCHUNKS
ROWS_PER_W = TR_PER_W * TILE_R
SC_SPAN = NUM_WORKERS * ROWS_PER_W
RU = 2
NBUF = 2

TC_BLOCK = 4000
TC_START_BLK = 11
TC_NBLKS = SCAN // TC_BLOCK - TC_START_BLK


def _scan_chunk(buf, ms, gs, ivec):
    def body(i, carry):
        ms, gs, ivec = carry
        ms, gs = list(ms), list(gs)
        base = pl.multiple_of(i * RU, RU)
        for p in range(RU):
            for cg in range(CGROUPS):
                u = p * CGROUPS + cg
                v = buf[base + p, pl.ds(cg * LANES, LANES)]
                take = v > ms[u]
                ms[u] = jnp.where(take, v, ms[u])
                gs[u] = jnp.where(take, ivec, gs[u])
        return tuple(ms), tuple(gs), ivec + 1

    ms, gs, ivec = lax.fori_loop(
        0, CHUNK_ROWS // RU, body, (tuple(ms), tuple(gs), ivec)
    )
    return list(ms), list(gs), ivec


def _sc_argmax_body(emb_hbm, out_hbm, buf0, buf1, stage, sem0, sem1):
    cid = lax.axis_index("c")
    sid = lax.axis_index("s")
    w = cid * NUM_SUBCORES + sid
    row_base = pl.multiple_of(w * ROWS_PER_W, TILE_R)

    neg_inf = jnp.full((LANES,), -jnp.inf, jnp.float32)
    zero = jnp.zeros((LANES,), jnp.int32)
    bufs = (buf0, buf1)
    sems = (sem0, sem1)

    def start(k):
        roff = pl.multiple_of(row_base + k * CHUNK_ROWS, TILE_R)
        return pltpu.async_copy(
            emb_hbm.at[pl.ds(roff, CHUNK_ROWS)], bufs[k % NBUF], sems[k % NBUF]
        )

    handles = {0: start(0)}
    nchains = CGROUPS * RU
    ms = [neg_inf] * nchains
    gs = [zero] * nchains
    ivec = zero
    for k in range(NCHUNKS):
        if k + 1 < NCHUNKS:
            handles[k + 1] = start(k + 1)
        handles[k].wait()
        ms, gs, ivec = _scan_chunk(bufs[k % NBUF], ms, gs, ivec)

    for p in range(RU):
        for cg in range(CGROUPS):
            u = p * CGROUPS + cg
            gs[u] = row_base + gs[u] * RU + p
    mm = [ms[cg] for cg in range(CGROUPS)]
    gg = [gs[cg] for cg in range(CGROUPS)]
    for p in range(1, RU):
        for cg in range(CGROUPS):
            u = p * CGROUPS + cg
            m2, g2 = ms[u], gs[u]
            better = (m2 > mm[cg]) | ((m2 == mm[cg]) & (g2 < gg[cg]))
            mm[cg] = jnp.where(better, m2, mm[cg])
            gg[cg] = jnp.where(better, g2, gg[cg])

    for cg in range(CGROUPS):
        stage[0, cg, :] = mm[cg]
        stage[1, cg, :] = gg[cg].astype(jnp.float32)
    pltpu.sync_copy(stage, out_hbm.at[w])


def _tc_argmax_body(x_ref, out_ref):
    i = pl.program_id(0)
    x = x_ref[...]
    m = jnp.max(x, axis=0)
    ridx = lax.broadcasted_iota(jnp.int32, (TC_BLOCK, ROWS), 0)
    g = jnp.min(jnp.where(x == m[None, :], ridx, jnp.int32(0x7FFFFFFF)), axis=0)
    row0 = (TC_START_BLK + i) * TC_BLOCK
    gf = (g + row0).astype(jnp.float32)

    @pl.when(i == 0)
    def _():
        out_ref[0, :] = m
        out_ref[1, :] = gf

    @pl.when(i > 0)
    def _():
        pm = out_ref[0, :]
        better = m > pm
        out_ref[0, :] = jnp.where(better, m, pm)
        out_ref[1, :] = jnp.where(better, gf, out_ref[1, :])


@jax.jit
def kernel(embedding):
    emb_t = embedding.T

    sc_call = functools.partial(
        pl.kernel,
        mesh=plsc.VectorSubcoreMesh(core_axis_name="c", subcore_axis_name="s"),
        out_type=jax.ShapeDtypeStruct((NUM_WORKERS, 2, CGROUPS, LANES), jnp.float32),
        scratch_types=[
            pltpu.VMEM((CHUNK_ROWS, ROWS), jnp.float32),
            pltpu.VMEM((CHUNK_ROWS, ROWS), jnp.float32),
            pltpu.VMEM((2, CGROUPS, LANES), jnp.float32),
            pltpu.SemaphoreType.DMA,
            pltpu.SemaphoreType.DMA,
        ],
    )(_sc_argmax_body)
    sc_out = sc_call(emb_t)

    tc_out = pl.pallas_call(
        _tc_argmax_body,
        grid=(TC_NBLKS,),
        in_specs=[
            pl.BlockSpec((TC_BLOCK, ROWS), lambda i: (TC_START_BLK + i, 0))
        ],
        out_specs=pl.BlockSpec((2, ROWS), lambda i: (0, 0)),
        out_shape=jax.ShapeDtypeStruct((2, ROWS), jnp.float32),
        compiler_params=pltpu.CompilerParams(
            dimension_semantics=("arbitrary",)
        ),
    )(emb_t)

    msc = sc_out[:, 0].reshape(NUM_WORKERS, ROWS)
    gsc = sc_out[:, 1].reshape(NUM_WORKERS, ROWS)
    wbest = jnp.argmax(msc, axis=0)
    m = jnp.take_along_axis(msc, wbest[None, :], axis=0)[0]
    g = jnp.take_along_axis(gsc, wbest[None, :], axis=0)[0]
    mt, gt = tc_out[0], tc_out[1]
    taket = (mt > m) | ((mt == m) & (gt < g))
    return jnp.where(taket, gt, g).astype(jnp.int32)
